# baseline (speedup 1.0000x reference)
import functools
import math

import jax
import jax.numpy as jnp
from jax import lax
from jax.experimental import pallas as pl
from jax.experimental.pallas import tpu as pltpu
from jax.experimental.pallas import tpu_sc as plsc

ROWS = 16384
COLS = 1000
NBINS = 30
TOT = float(ROWS)

R_SC = 8192
R_TC = ROWS - R_SC
C_SC = 896
C_ST = COLS - C_SC
BR = 256
G_TC = R_TC // BR
BS = 1024
G_ST = R_SC // BS

NC, NS, L = 2, 16, 16
NW = NC * NS
RPW = R_SC // NW
CR = 8
NT = C_SC // 128
NCH = RPW // CR
NI = CR * NT

_LOG1P = (1.4158017493e-04, 9.9542666178e-01, -4.6407070110e-01,
          2.1640858368e-01, -5.4862311289e-02)
K = 4

_THRESH = tuple(math.log((NBINS - i) / i) for i in range(1, NBINS))

_mesh = plsc.VectorSubcoreMesh(core_axis_name="c", subcore_axis_name="s")


@functools.partial(
    pl.kernel,
    mesh=_mesh,
    compiler_params=pltpu.CompilerParams(needs_layout_passes=False),
    out_type=[
        jax.ShapeDtypeStruct((NW, NBINS, C_SC), jnp.float32),
        jax.ShapeDtypeStruct((NW, NBINS, C_SC), jnp.float32),
    ],
    scratch_types=[
        pltpu.VMEM((NBINS, C_SC), jnp.float32),
        pltpu.VMEM((NBINS, C_SC), jnp.float32),
        pltpu.VMEM((NI, 128), jnp.float32),
        pltpu.VMEM((NI, 128), jnp.float32),
        pltpu.VMEM((NI, 128), jnp.int32),
        pltpu.SemaphoreType.DMA,
        pltpu.SemaphoreType.DMA,
    ],
)
def _sc_hist(v_hbm, cnt_out, sum_out,
             cnt_v, sum_v, pb0, pb1, jtab, sp0, sp1):
    wid = lax.axis_index("s") * NC + lax.axis_index("c")
    row_base = wid * RPW

    zeros = jnp.zeros((L,), jnp.float32)
    ones = jnp.full((L,), 1.0, jnp.float32)

    def zbody(r, carry):
        for cc in range(C_SC // L):
            cnt_v[r, pl.ds(cc * L, L)] = zeros
            sum_v[r, pl.ds(cc * L, L)] = zeros
        return carry

    lax.fori_loop(0, NBINS, zbody, 0)

    iota16 = lax.iota(jnp.int32, L)
    for ct in range(NT):
        cvs = [iota16 + (ct * 128 + gg * L) for gg in range(128 // L)]

        def jbody(r, carry, ct=ct, cvs=cvs):
            for gg in range(128 // L):
                jtab[ct * CR + r, pl.ds(gg * L, L)] = cvs[gg]
            return carry

        lax.fori_loop(0, CR, jbody, 0)

    def copies(pb, sp, c):
        r0 = row_base + c * CR
        return [
            pltpu.make_async_copy(
                v_hbm.at[pl.ds(r0, CR), pl.ds(ct * 128, 128)],
                pb.at[pl.ds(ct * CR, CR), :], sp)
            for ct in range(NT)
        ]

    def start(pb, sp, c):
        for cp in copies(pb, sp, c):
            cp.start()

    def wait(pb, sp, c):
        for cp in copies(pb, sp, c):
            cp.wait()

    def process(pb):
        def ibody(i, carry):
            for half in range(128 // (K * L)):
                ks = range(K)
                base = half * K * L
                v = [pb[i, pl.ds(base + k * L, L)] for k in ks]
                j = [jtab[i, pl.ds(base + k * L, L)] for k in ks]
                vi = [lax.bitcast_convert_type(v[k], jnp.int32) for k in ks]
                av = [jnp.abs(v[k]) for k in ks]
                nav = [lax.bitcast_convert_type(vi[k] | jnp.int32(-2**31),
                                                jnp.float32) for k in ks]
                e = [jnp.exp(nav[k]) for k in ks]
                d = [1.0 + e[k] for k in ks]
                r = [1.0 / d[k] for k in ks]
                rr = [r[k] * float(NBINS) for k in ks]
                g30 = [jnp.where(v[k] >= 0.0, float(NBINS) - rr[k], rr[k])
                       for k in ks]
                g30 = [jnp.minimum(g30[k], float(NBINS) - 0.5) for k in ks]
                b = [g30[k].astype(jnp.int32) for k in ks]
                relu = [(av[k] - v[k]) * 0.5 for k in ks]
                p = [jnp.full((L,), _LOG1P[4], jnp.float32)] * K
                for cc in _LOG1P[3::-1]:
                    p = [p[k] * e[k] + cc for k in ks]
                bce = [relu[k] + p[k] for k in ks]
                for k in ks:
                    plsc.addupdate_scatter(cnt_v, [b[k], j[k]], ones)
                    plsc.addupdate_scatter(sum_v, [b[k], j[k]], bce[k])
            return carry

        lax.fori_loop(0, NI, ibody, 0)

    start(pb0, sp0, 0)
    start(pb1, sp1, 1)

    def pbody(pi, carry):
        c0 = 2 * pi
        wait(pb0, sp0, c0)
        process(pb0)

        @pl.when(c0 + 2 < NCH)
        def _():
            start(pb0, sp0, c0 + 2)

        wait(pb1, sp1, c0 + 1)
        process(pb1)

        @pl.when(c0 + 3 < NCH)
        def _():
            start(pb1, sp1, c0 + 3)

        return carry

    lax.fori_loop(0, NCH // 2, pbody, 0)

    pltpu.sync_copy(cnt_v, cnt_out.at[wid])
    pltpu.sync_copy(sum_v, sum_out.at[wid])


def _tc_hist_body(x_ref, t_ref, cnt_ref, sum_ref):
    x = x_ref[...]
    t = t_ref[...].astype(jnp.float32)
    v = (1.0 - 2.0 * t) * x
    bce = jnp.maximum(-v, 0.0) + jnp.log1p(jnp.exp(-jnp.abs(v)))
    prev_c = jnp.full((v.shape[1],), float(v.shape[0]), jnp.float32)
    prev_s = jnp.sum(bce, axis=0)
    cs, ss = [], []
    for th in _THRESH:
        m = (v <= th).astype(jnp.float32)
        ci = jnp.sum(m, axis=0)
        si = jnp.sum(m * bce, axis=0)
        cs.append(prev_c - ci)
        ss.append(prev_s - si)
        prev_c, prev_s = ci, si
    cs.append(prev_c)
    ss.append(prev_s)
    cnt_ref[0] = jnp.stack(cs)
    sum_ref[0] = jnp.stack(ss)


def _make_tc_hist(nrows, width, grid, row_off):
    return pl.pallas_call(
        _tc_hist_body,
        grid=(grid,),
        in_specs=[
            pl.BlockSpec((nrows, width), lambda i: (row_off + i, 0)),
            pl.BlockSpec((nrows, width), lambda i: (row_off + i, 0)),
        ],
        out_specs=[
            pl.BlockSpec((1, NBINS, width), lambda i: (i, 0, 0)),
            pl.BlockSpec((1, NBINS, width), lambda i: (i, 0, 0)),
        ],
        out_shape=[
            jax.ShapeDtypeStruct((grid, NBINS, width), jnp.float32),
            jax.ShapeDtypeStruct((grid, NBINS, width), jnp.float32),
        ],
    )


_tc_main = _make_tc_hist(BR, COLS, G_TC, R_SC // BR)
_tc_strip = _make_tc_hist(BS, C_ST, G_ST, 0)


def _finalize_body(csc_ref, ssc_ref, cm_ref, sm_ref, cst_ref, sst_ref, out_ref):
    csc = jnp.sum(csc_ref[...], axis=0)
    ssc = jnp.sum(ssc_ref[...], axis=0)
    c = jnp.sum(cm_ref[...], axis=0)
    s = jnp.sum(sm_ref[...], axis=0)
    cst = jnp.sum(cst_ref[...], axis=0)
    sst = jnp.sum(sst_ref[...], axis=0)
    c = c + jnp.concatenate([csc, cst], axis=1)
    s = s + jnp.concatenate([ssc, sst], axis=1)
    nz = c > 0.0
    w = jnp.where(nz, TOT / (0.25 * jnp.maximum(c, 1.0)), 0.0)
    n = jnp.maximum(jnp.sum(nz.astype(jnp.float32), axis=0), 1.0)
    colsum = jnp.sum(s * w, axis=0)
    loss = jnp.sum(colsum / n) / float(ROWS * COLS)
    out_ref[...] = jnp.broadcast_to(loss, (1, 1))


_finalize = pl.pallas_call(
    _finalize_body,
    out_shape=jax.ShapeDtypeStruct((1, 1), jnp.float32),
)


def kernel(preds, targets):
    v = preds * (1.0 - 2.0 * targets.astype(jnp.float32))
    cnt_sc, sum_sc = _sc_hist(v)
    cnt_tc, sum_tc = _tc_main(preds, targets)
    cnt_st, sum_st = _tc_strip(preds[:R_SC, C_SC:], targets[:R_SC, C_SC:])
    loss = _finalize(cnt_sc, sum_sc, cnt_tc, sum_tc, cnt_st, sum_st)
    return loss[0, 0]


# speedup vs baseline: 1.0712x; 1.0712x over previous
import functools
import math

import jax
import jax.numpy as jnp
from jax import lax
from jax.experimental import pallas as pl
from jax.experimental.pallas import tpu as pltpu
from jax.experimental.pallas import tpu_sc as plsc

ROWS = 16384
COLS = 1000
NBINS = 30
TOT = float(ROWS)

R_SC = 8192
R_TC = ROWS - R_SC
C_SC = 896
C_ST = COLS - C_SC
BR = 256
G_TC = R_TC // BR
BS = 1024
G_ST = R_SC // BS

NC, NS, L = 2, 16, 16
NW = NC * NS
RPW = R_SC // NW
CR = 8
NT = C_SC // 128
NCH = RPW // CR
NI = CR * NT

_LOG1P = (1.4158017493e-04, 9.9542666178e-01, -4.6407070110e-01,
          2.1640858368e-01, -5.4862311289e-02)
K = 4

_THRESH = tuple(math.log((NBINS - i) / i) for i in range(1, NBINS))

_mesh = plsc.VectorSubcoreMesh(core_axis_name="c", subcore_axis_name="s")


@functools.partial(
    pl.kernel,
    mesh=_mesh,
    compiler_params=pltpu.CompilerParams(needs_layout_passes=False),
    out_type=[
        jax.ShapeDtypeStruct((NW, NBINS, C_SC), jnp.float32),
        jax.ShapeDtypeStruct((NW, NBINS, C_SC), jnp.float32),
    ],
    scratch_types=[
        pltpu.VMEM((NBINS, C_SC), jnp.float32),
        pltpu.VMEM((NBINS, C_SC), jnp.float32),
        pltpu.VMEM((NI, 128), jnp.float32),
        pltpu.VMEM((NI, 128), jnp.float32),
        pltpu.VMEM((NI, 128), jnp.int32),
        pltpu.SemaphoreType.DMA,
        pltpu.SemaphoreType.DMA,
    ],
)
def _sc_hist(v_hbm, cnt_out, sum_out,
             cnt_v, sum_v, pb0, pb1, jtab, sp0, sp1):
    wid = lax.axis_index("s") * NC + lax.axis_index("c")
    row_base = wid * RPW

    zeros = jnp.zeros((L,), jnp.float32)
    ones = jnp.full((L,), 1.0, jnp.float32)

    def zbody(r, carry):
        for cc in range(C_SC // L):
            cnt_v[r, pl.ds(cc * L, L)] = zeros
            sum_v[r, pl.ds(cc * L, L)] = zeros
        return carry

    lax.fori_loop(0, NBINS, zbody, 0)

    iota16 = lax.iota(jnp.int32, L)
    for ct in range(NT):
        cvs = [iota16 + (ct * 128 + gg * L) for gg in range(128 // L)]

        def jbody(r, carry, ct=ct, cvs=cvs):
            for gg in range(128 // L):
                jtab[ct * CR + r, pl.ds(gg * L, L)] = cvs[gg]
            return carry

        lax.fori_loop(0, CR, jbody, 0)

    def copies(pb, sp, c):
        r0 = row_base + c * CR
        return [
            pltpu.make_async_copy(
                v_hbm.at[pl.ds(r0, CR), pl.ds(ct * 128, 128)],
                pb.at[pl.ds(ct * CR, CR), :], sp)
            for ct in range(NT)
        ]

    def start(pb, sp, c):
        for cp in copies(pb, sp, c):
            cp.start()

    def wait(pb, sp, c):
        for cp in copies(pb, sp, c):
            cp.wait()

    def process(pb):
        def ibody(i, carry):
            for half in range(128 // (K * L)):
                ks = range(K)
                base = half * K * L
                v = [pb[i, pl.ds(base + k * L, L)] for k in ks]
                j = [jtab[i, pl.ds(base + k * L, L)] for k in ks]
                vi = [lax.bitcast_convert_type(v[k], jnp.int32) for k in ks]
                av = [jnp.abs(v[k]) for k in ks]
                nav = [lax.bitcast_convert_type(vi[k] | jnp.int32(-2**31),
                                                jnp.float32) for k in ks]
                e = [jnp.exp(nav[k]) for k in ks]
                d = [1.0 + e[k] for k in ks]
                r = [1.0 / d[k] for k in ks]
                rr = [r[k] * float(NBINS) for k in ks]
                g30 = [jnp.where(v[k] >= 0.0, float(NBINS) - rr[k], rr[k])
                       for k in ks]
                g30 = [jnp.minimum(g30[k], float(NBINS) - 0.5) for k in ks]
                b = [g30[k].astype(jnp.int32) for k in ks]
                relu = [(av[k] - v[k]) * 0.5 for k in ks]
                p = [jnp.full((L,), _LOG1P[4], jnp.float32)] * K
                for cc in _LOG1P[3::-1]:
                    p = [p[k] * e[k] + cc for k in ks]
                bce = [relu[k] + p[k] for k in ks]
                for k in ks:
                    plsc.addupdate_scatter(cnt_v, [b[k], j[k]], ones)
                    plsc.addupdate_scatter(sum_v, [b[k], j[k]], bce[k])
            return carry

        lax.fori_loop(0, NI, ibody, 0)

    start(pb0, sp0, 0)
    start(pb1, sp1, 1)

    def pbody(pi, carry):
        c0 = 2 * pi
        wait(pb0, sp0, c0)
        process(pb0)

        @pl.when(c0 + 2 < NCH)
        def _():
            start(pb0, sp0, c0 + 2)

        wait(pb1, sp1, c0 + 1)
        process(pb1)

        @pl.when(c0 + 3 < NCH)
        def _():
            start(pb1, sp1, c0 + 3)

        return carry

    lax.fori_loop(0, NCH // 2, pbody, 0)

    pltpu.sync_copy(cnt_v, cnt_out.at[wid])
    pltpu.sync_copy(sum_v, sum_out.at[wid])


def _tc_hist_body(x_ref, t_ref, cnt_ref, sum_ref):
    x = x_ref[...]
    t = t_ref[...].astype(jnp.float32)
    v = (1.0 - 2.0 * t) * x
    bce = jnp.maximum(-v, 0.0) + jnp.log1p(jnp.exp(-jnp.abs(v)))
    prev_c = jnp.full((v.shape[1],), float(v.shape[0]), jnp.float32)
    prev_s = jnp.sum(bce, axis=0)
    cs, ss = [], []
    for th in _THRESH:
        m = (v <= th).astype(jnp.float32)
        ci = jnp.sum(m, axis=0)
        si = jnp.sum(m * bce, axis=0)
        cs.append(prev_c - ci)
        ss.append(prev_s - si)
        prev_c, prev_s = ci, si
    cs.append(prev_c)
    ss.append(prev_s)
    cnt_ref[0] = jnp.stack(cs)
    sum_ref[0] = jnp.stack(ss)


def _make_tc_hist(nrows, width, grid, row_off):
    return pl.pallas_call(
        _tc_hist_body,
        grid=(grid,),
        in_specs=[
            pl.BlockSpec((nrows, width), lambda i: (row_off + i, 0)),
            pl.BlockSpec((nrows, width), lambda i: (row_off + i, 0)),
        ],
        out_specs=[
            pl.BlockSpec((1, NBINS, width), lambda i: (i, 0, 0)),
            pl.BlockSpec((1, NBINS, width), lambda i: (i, 0, 0)),
        ],
        out_shape=[
            jax.ShapeDtypeStruct((grid, NBINS, width), jnp.float32),
            jax.ShapeDtypeStruct((grid, NBINS, width), jnp.float32),
        ],
    )


_tc_main = _make_tc_hist(BR, COLS, G_TC, R_SC // BR)
_tc_strip = _make_tc_hist(BS, C_ST, G_ST, 0)


def _finalize_body(csc_ref, ssc_ref, cm_ref, sm_ref, cst_ref, sst_ref, out_ref):
    csc = jnp.sum(csc_ref[...], axis=0)
    ssc = jnp.sum(ssc_ref[...], axis=0)
    c = jnp.sum(cm_ref[...], axis=0)
    s = jnp.sum(sm_ref[...], axis=0)
    cst = jnp.sum(cst_ref[...], axis=0)
    sst = jnp.sum(sst_ref[...], axis=0)
    c = c + jnp.concatenate([csc, cst], axis=1)
    s = s + jnp.concatenate([ssc, sst], axis=1)
    nz = c > 0.0
    w = jnp.where(nz, TOT / (0.25 * jnp.maximum(c, 1.0)), 0.0)
    n = jnp.maximum(jnp.sum(nz.astype(jnp.float32), axis=0), 1.0)
    colsum = jnp.sum(s * w, axis=0)
    loss = jnp.sum(colsum / n) / float(ROWS * COLS)
    out_ref[...] = jnp.broadcast_to(loss, (1, 1))


_finalize = pl.pallas_call(
    _finalize_body,
    out_shape=jax.ShapeDtypeStruct((1, 1), jnp.float32),
)


def kernel(preds, targets):
    v = preds[:R_SC] * (1.0 - 2.0 * targets[:R_SC].astype(jnp.float32))
    cnt_sc, sum_sc = _sc_hist(v)
    cnt_tc, sum_tc = _tc_main(preds, targets)
    cnt_st, sum_st = _tc_strip(preds[:R_SC, C_SC:], targets[:R_SC, C_SC:])
    loss = _finalize(cnt_sc, sum_sc, cnt_tc, sum_tc, cnt_st, sum_st)
    return loss[0, 0]


# speedup vs baseline: 1.1729x; 1.0950x over previous
import functools
import math

import jax
import jax.numpy as jnp
from jax import lax
from jax.experimental import pallas as pl
from jax.experimental.pallas import tpu as pltpu
from jax.experimental.pallas import tpu_sc as plsc

ROWS = 16384
COLS = 1000
NBINS = 30
TOT = float(ROWS)

R_SC = 10240
R_TC = ROWS - R_SC
C_SC = 896
C_ST = COLS - C_SC
BR = 256
G_TC = R_TC // BR
BS = 1024
G_ST = R_SC // BS

NC, NS, L = 2, 16, 16
NW = NC * NS
RPW = R_SC // NW
CR = 8
NT = C_SC // 128
NCH = RPW // CR
NI = CR * NT

_LOG1P = (1.4158017493e-04, 9.9542666178e-01, -4.6407070110e-01,
          2.1640858368e-01, -5.4862311289e-02)
K = 4

_THRESH = tuple(math.log((NBINS - i) / i) for i in range(1, NBINS))

_mesh = plsc.VectorSubcoreMesh(core_axis_name="c", subcore_axis_name="s")


@functools.partial(
    pl.kernel,
    mesh=_mesh,
    compiler_params=pltpu.CompilerParams(needs_layout_passes=False),
    out_type=[
        jax.ShapeDtypeStruct((NW, NBINS, C_SC), jnp.float32),
        jax.ShapeDtypeStruct((NW, NBINS, C_SC), jnp.float32),
    ],
    scratch_types=[
        pltpu.VMEM((NBINS, C_SC), jnp.float32),
        pltpu.VMEM((NBINS, C_SC), jnp.float32),
        pltpu.VMEM((NI, 128), jnp.float32),
        pltpu.VMEM((NI, 128), jnp.float32),
        pltpu.VMEM((NI, 128), jnp.int32),
        pltpu.SemaphoreType.DMA,
        pltpu.SemaphoreType.DMA,
    ],
)
def _sc_hist(v_hbm, cnt_out, sum_out,
             cnt_v, sum_v, pb0, pb1, jtab, sp0, sp1):
    wid = lax.axis_index("s") * NC + lax.axis_index("c")
    row_base = wid * RPW

    zeros = jnp.zeros((L,), jnp.float32)
    ones = jnp.full((L,), 1.0, jnp.float32)

    def zbody(r, carry):
        for cc in range(C_SC // L):
            cnt_v[r, pl.ds(cc * L, L)] = zeros
            sum_v[r, pl.ds(cc * L, L)] = zeros
        return carry

    lax.fori_loop(0, NBINS, zbody, 0)

    iota16 = lax.iota(jnp.int32, L)
    for ct in range(NT):
        cvs = [iota16 + (ct * 128 + gg * L) for gg in range(128 // L)]

        def jbody(r, carry, ct=ct, cvs=cvs):
            for gg in range(128 // L):
                jtab[ct * CR + r, pl.ds(gg * L, L)] = cvs[gg]
            return carry

        lax.fori_loop(0, CR, jbody, 0)

    def copies(pb, sp, c):
        r0 = row_base + c * CR
        return [
            pltpu.make_async_copy(
                v_hbm.at[pl.ds(r0, CR), pl.ds(ct * 128, 128)],
                pb.at[pl.ds(ct * CR, CR), :], sp)
            for ct in range(NT)
        ]

    def start(pb, sp, c):
        for cp in copies(pb, sp, c):
            cp.start()

    def wait(pb, sp, c):
        for cp in copies(pb, sp, c):
            cp.wait()

    def process(pb):
        def ibody(i, carry):
            for half in range(128 // (K * L)):
                ks = range(K)
                base = half * K * L
                v = [pb[i, pl.ds(base + k * L, L)] for k in ks]
                j = [jtab[i, pl.ds(base + k * L, L)] for k in ks]
                vi = [lax.bitcast_convert_type(v[k], jnp.int32) for k in ks]
                av = [jnp.abs(v[k]) for k in ks]
                nav = [lax.bitcast_convert_type(vi[k] | jnp.int32(-2**31),
                                                jnp.float32) for k in ks]
                e = [jnp.exp(nav[k]) for k in ks]
                d = [1.0 + e[k] for k in ks]
                r = [1.0 / d[k] for k in ks]
                rr = [r[k] * float(NBINS) for k in ks]
                g30 = [jnp.where(v[k] >= 0.0, float(NBINS) - rr[k], rr[k])
                       for k in ks]
                g30 = [jnp.minimum(g30[k], float(NBINS) - 0.5) for k in ks]
                b = [g30[k].astype(jnp.int32) for k in ks]
                relu = [(av[k] - v[k]) * 0.5 for k in ks]
                p = [jnp.full((L,), _LOG1P[4], jnp.float32)] * K
                for cc in _LOG1P[3::-1]:
                    p = [p[k] * e[k] + cc for k in ks]
                bce = [relu[k] + p[k] for k in ks]
                for k in ks:
                    plsc.addupdate_scatter(cnt_v, [b[k], j[k]], ones)
                    plsc.addupdate_scatter(sum_v, [b[k], j[k]], bce[k])
            return carry

        lax.fori_loop(0, NI, ibody, 0, unroll=2)

    start(pb0, sp0, 0)
    start(pb1, sp1, 1)

    def pbody(pi, carry):
        c0 = 2 * pi
        wait(pb0, sp0, c0)
        process(pb0)

        @pl.when(c0 + 2 < NCH)
        def _():
            start(pb0, sp0, c0 + 2)

        wait(pb1, sp1, c0 + 1)
        process(pb1)

        @pl.when(c0 + 3 < NCH)
        def _():
            start(pb1, sp1, c0 + 3)

        return carry

    lax.fori_loop(0, NCH // 2, pbody, 0)

    pltpu.sync_copy(cnt_v, cnt_out.at[wid])
    pltpu.sync_copy(sum_v, sum_out.at[wid])


def _tc_hist_body(v_ref, cnt_ref, sum_ref):
    v = v_ref[...]
    bce = jnp.maximum(-v, 0.0) + jnp.log1p(jnp.exp(-jnp.abs(v)))
    prev_c = jnp.full((v.shape[1],), float(v.shape[0]), jnp.float32)
    prev_s = jnp.sum(bce, axis=0)
    cs, ss = [], []
    for th in _THRESH:
        m = (v <= th).astype(jnp.float32)
        ci = jnp.sum(m, axis=0)
        si = jnp.sum(m * bce, axis=0)
        cs.append(prev_c - ci)
        ss.append(prev_s - si)
        prev_c, prev_s = ci, si
    cs.append(prev_c)
    ss.append(prev_s)
    cnt_ref[0] = jnp.stack(cs)
    sum_ref[0] = jnp.stack(ss)


def _make_tc_hist(nrows, width, grid, row_off):
    return pl.pallas_call(
        _tc_hist_body,
        grid=(grid,),
        in_specs=[pl.BlockSpec((nrows, width), lambda i: (row_off + i, 0))],
        out_specs=[
            pl.BlockSpec((1, NBINS, width), lambda i: (i, 0, 0)),
            pl.BlockSpec((1, NBINS, width), lambda i: (i, 0, 0)),
        ],
        out_shape=[
            jax.ShapeDtypeStruct((grid, NBINS, width), jnp.float32),
            jax.ShapeDtypeStruct((grid, NBINS, width), jnp.float32),
        ],
    )


_tc_main = _make_tc_hist(BR, COLS, G_TC, R_SC // BR)
_tc_strip = _make_tc_hist(BS, C_ST, G_ST, 0)


def _finalize_body(csc_ref, ssc_ref, cm_ref, sm_ref, cst_ref, sst_ref, out_ref):
    csc = jnp.sum(csc_ref[...], axis=0)
    ssc = jnp.sum(ssc_ref[...], axis=0)
    c = jnp.sum(cm_ref[...], axis=0)
    s = jnp.sum(sm_ref[...], axis=0)
    cst = jnp.sum(cst_ref[...], axis=0)
    sst = jnp.sum(sst_ref[...], axis=0)
    c = c + jnp.concatenate([csc, cst], axis=1)
    s = s + jnp.concatenate([ssc, sst], axis=1)
    nz = c > 0.0
    w = jnp.where(nz, TOT / (0.25 * jnp.maximum(c, 1.0)), 0.0)
    n = jnp.maximum(jnp.sum(nz.astype(jnp.float32), axis=0), 1.0)
    colsum = jnp.sum(s * w, axis=0)
    loss = jnp.sum(colsum / n) / float(ROWS * COLS)
    out_ref[...] = jnp.broadcast_to(loss, (1, 1))


_finalize = pl.pallas_call(
    _finalize_body,
    out_shape=jax.ShapeDtypeStruct((1, 1), jnp.float32),
)


def kernel(preds, targets):
    v = preds * (1.0 - 2.0 * targets.astype(jnp.float32))
    cnt_sc, sum_sc = _sc_hist(v)
    cnt_tc, sum_tc = _tc_main(v)
    cnt_st, sum_st = _tc_strip(v[:R_SC, C_SC:])
    loss = _finalize(cnt_sc, sum_sc, cnt_tc, sum_tc, cnt_st, sum_st)
    return loss[0, 0]


# speedup vs baseline: 1.1735x; 1.0005x over previous
import functools
import math

import jax
import jax.numpy as jnp
from jax import lax
from jax.experimental import pallas as pl
from jax.experimental.pallas import tpu as pltpu
from jax.experimental.pallas import tpu_sc as plsc

ROWS = 16384
COLS = 1000
NBINS = 30
TOT = float(ROWS)

R_SC = 10240
R_TC = ROWS - R_SC
C_SC = 896
C_ST = COLS - C_SC
BR = 256
G_TC = R_TC // BR
BS = 2048
G_ST = R_SC // BS

NC, NS, L = 2, 16, 16
NW = NC * NS
RPW = R_SC // NW
CR = 8
NT = C_SC // 128
NCH = RPW // CR
NI = CR * NT

_LOG1P = (1.4158017493e-04, 9.9542666178e-01, -4.6407070110e-01,
          2.1640858368e-01, -5.4862311289e-02)
K = 4

_THRESH = tuple(math.log((NBINS - i) / i) for i in range(1, NBINS))

_mesh = plsc.VectorSubcoreMesh(core_axis_name="c", subcore_axis_name="s")


@functools.partial(
    pl.kernel,
    mesh=_mesh,
    compiler_params=pltpu.CompilerParams(needs_layout_passes=False),
    out_type=[
        jax.ShapeDtypeStruct((NW, NBINS, C_SC), jnp.float32),
        jax.ShapeDtypeStruct((NW, NBINS, C_SC), jnp.float32),
    ],
    scratch_types=[
        pltpu.VMEM((NBINS, C_SC), jnp.float32),
        pltpu.VMEM((NBINS, C_SC), jnp.float32),
        pltpu.VMEM((NI, 128), jnp.float32),
        pltpu.VMEM((NI, 128), jnp.float32),
        pltpu.VMEM((NI, 128), jnp.int32),
        pltpu.SemaphoreType.DMA,
        pltpu.SemaphoreType.DMA,
    ],
)
def _sc_hist(v_hbm, cnt_out, sum_out,
             cnt_v, sum_v, pb0, pb1, jtab, sp0, sp1):
    wid = lax.axis_index("s") * NC + lax.axis_index("c")
    row_base = wid * RPW

    zeros = jnp.zeros((L,), jnp.float32)
    ones = jnp.full((L,), 1.0, jnp.float32)

    def zbody(r, carry):
        for cc in range(C_SC // L):
            cnt_v[r, pl.ds(cc * L, L)] = zeros
            sum_v[r, pl.ds(cc * L, L)] = zeros
        return carry

    lax.fori_loop(0, NBINS, zbody, 0)

    iota16 = lax.iota(jnp.int32, L)
    for ct in range(NT):
        cvs = [iota16 + (ct * 128 + gg * L) for gg in range(128 // L)]

        def jbody(r, carry, ct=ct, cvs=cvs):
            for gg in range(128 // L):
                jtab[ct * CR + r, pl.ds(gg * L, L)] = cvs[gg]
            return carry

        lax.fori_loop(0, CR, jbody, 0)

    def copies(pb, sp, c):
        r0 = row_base + c * CR
        return [
            pltpu.make_async_copy(
                v_hbm.at[pl.ds(r0, CR), pl.ds(ct * 128, 128)],
                pb.at[pl.ds(ct * CR, CR), :], sp)
            for ct in range(NT)
        ]

    def start(pb, sp, c):
        for cp in copies(pb, sp, c):
            cp.start()

    def wait(pb, sp, c):
        for cp in copies(pb, sp, c):
            cp.wait()

    def process(pb):
        def ibody(i, carry):
            for half in range(128 // (K * L)):
                ks = range(K)
                base = half * K * L
                v = [pb[i, pl.ds(base + k * L, L)] for k in ks]
                j = [jtab[i, pl.ds(base + k * L, L)] for k in ks]
                vi = [lax.bitcast_convert_type(v[k], jnp.int32) for k in ks]
                av = [jnp.abs(v[k]) for k in ks]
                nav = [lax.bitcast_convert_type(vi[k] | jnp.int32(-2**31),
                                                jnp.float32) for k in ks]
                e = [jnp.exp(nav[k]) for k in ks]
                d = [1.0 + e[k] for k in ks]
                r = [1.0 / d[k] for k in ks]
                rr = [r[k] * float(NBINS) for k in ks]
                g30 = [jnp.where(v[k] >= 0.0, float(NBINS) - rr[k], rr[k])
                       for k in ks]
                g30 = [jnp.minimum(g30[k], float(NBINS) - 0.5) for k in ks]
                b = [g30[k].astype(jnp.int32) for k in ks]
                relu = [(av[k] - v[k]) * 0.5 for k in ks]
                p = [jnp.full((L,), _LOG1P[4], jnp.float32)] * K
                for cc in _LOG1P[3::-1]:
                    p = [p[k] * e[k] + cc for k in ks]
                bce = [relu[k] + p[k] for k in ks]
                for k in ks:
                    plsc.addupdate_scatter(cnt_v, [b[k], j[k]], ones)
                    plsc.addupdate_scatter(sum_v, [b[k], j[k]], bce[k])
            return carry

        lax.fori_loop(0, NI, ibody, 0, unroll=2)

    start(pb0, sp0, 0)
    start(pb1, sp1, 1)

    def pbody(pi, carry):
        c0 = 2 * pi
        wait(pb0, sp0, c0)
        process(pb0)

        @pl.when(c0 + 2 < NCH)
        def _():
            start(pb0, sp0, c0 + 2)

        wait(pb1, sp1, c0 + 1)
        process(pb1)

        @pl.when(c0 + 3 < NCH)
        def _():
            start(pb1, sp1, c0 + 3)

        return carry

    lax.fori_loop(0, NCH // 2, pbody, 0)

    pltpu.sync_copy(cnt_v, cnt_out.at[wid])
    pltpu.sync_copy(sum_v, sum_out.at[wid])


def _tc_hist_body(v_ref, cnt_ref, sum_ref):
    v = v_ref[...]
    bce = jnp.maximum(-v, 0.0) + jnp.log1p(jnp.exp(-jnp.abs(v)))
    prev_c = jnp.full((v.shape[1],), float(v.shape[0]), jnp.float32)
    prev_s = jnp.sum(bce, axis=0)
    cs, ss = [], []
    for th in _THRESH:
        m = (v <= th).astype(jnp.float32)
        ci = jnp.sum(m, axis=0)
        si = jnp.sum(m * bce, axis=0)
        cs.append(prev_c - ci)
        ss.append(prev_s - si)
        prev_c, prev_s = ci, si
    cs.append(prev_c)
    ss.append(prev_s)
    cnt_ref[0] = jnp.stack(cs)
    sum_ref[0] = jnp.stack(ss)


def _make_tc_hist(nrows, width, grid, row_off):
    return pl.pallas_call(
        _tc_hist_body,
        grid=(grid,),
        in_specs=[pl.BlockSpec((nrows, width), lambda i: (row_off + i, 0))],
        out_specs=[
            pl.BlockSpec((1, NBINS, width), lambda i: (i, 0, 0)),
            pl.BlockSpec((1, NBINS, width), lambda i: (i, 0, 0)),
        ],
        out_shape=[
            jax.ShapeDtypeStruct((grid, NBINS, width), jnp.float32),
            jax.ShapeDtypeStruct((grid, NBINS, width), jnp.float32),
        ],
    )


_tc_main = _make_tc_hist(BR, COLS, G_TC, R_SC // BR)
_tc_strip = _make_tc_hist(BS, C_ST, G_ST, 0)


def _finalize_body(csc_ref, ssc_ref, cm_ref, sm_ref, cst_ref, sst_ref, out_ref):
    csc = jnp.sum(csc_ref[...], axis=0)
    ssc = jnp.sum(ssc_ref[...], axis=0)
    c = jnp.sum(cm_ref[...], axis=0)
    s = jnp.sum(sm_ref[...], axis=0)
    cst = jnp.sum(cst_ref[...], axis=0)
    sst = jnp.sum(sst_ref[...], axis=0)
    c = c + jnp.concatenate([csc, cst], axis=1)
    s = s + jnp.concatenate([ssc, sst], axis=1)
    nz = c > 0.0
    w = jnp.where(nz, TOT / (0.25 * jnp.maximum(c, 1.0)), 0.0)
    n = jnp.maximum(jnp.sum(nz.astype(jnp.float32), axis=0), 1.0)
    colsum = jnp.sum(s * w, axis=0)
    loss = jnp.sum(colsum / n) / float(ROWS * COLS)
    out_ref[...] = jnp.broadcast_to(loss, (1, 1))


_finalize = pl.pallas_call(
    _finalize_body,
    out_shape=jax.ShapeDtypeStruct((1, 1), jnp.float32),
)


def kernel(preds, targets):
    v = preds * (1.0 - 2.0 * targets.astype(jnp.float32))
    cnt_sc, sum_sc = _sc_hist(v)
    cnt_tc, sum_tc = _tc_main(v)
    cnt_st, sum_st = _tc_strip(v[:R_SC, C_SC:])
    loss = _finalize(cnt_sc, sum_sc, cnt_tc, sum_tc, cnt_st, sum_st)
    return loss[0, 0]


# speedup vs baseline: 1.1738x; 1.0002x over previous
import functools
import math

import jax
import jax.numpy as jnp
from jax import lax
from jax.experimental import pallas as pl
from jax.experimental.pallas import tpu as pltpu
from jax.experimental.pallas import tpu_sc as plsc

ROWS = 16384
COLS = 1000
NBINS = 30
TOT = float(ROWS)

R_SC = 10240
R_TC = ROWS - R_SC
C_SC = 896
C_ST = COLS - C_SC
BR = 256
G_TC = R_TC // BR
BS = 2048
G_ST = R_SC // BS

NC, NS, L = 2, 16, 16
NW = NC * NS
RPW = R_SC // NW
CR = 8
NT = C_SC // 128
NCH = RPW // CR
NI = CR * NT

_LOG1P = (1.4158017493e-04, 9.9542666178e-01, -4.6407070110e-01,
          2.1640858368e-01, -5.4862311289e-02)
K = 8

_THRESH = tuple(math.log((NBINS - i) / i) for i in range(1, NBINS))

_mesh = plsc.VectorSubcoreMesh(core_axis_name="c", subcore_axis_name="s")


@functools.partial(
    pl.kernel,
    mesh=_mesh,
    compiler_params=pltpu.CompilerParams(needs_layout_passes=False),
    out_type=[
        jax.ShapeDtypeStruct((NW, NBINS, C_SC), jnp.float32),
        jax.ShapeDtypeStruct((NW, NBINS, C_SC), jnp.float32),
    ],
    scratch_types=[
        pltpu.VMEM((NBINS, C_SC), jnp.float32),
        pltpu.VMEM((NBINS, C_SC), jnp.float32),
        pltpu.VMEM((NI, 128), jnp.float32),
        pltpu.VMEM((NI, 128), jnp.float32),
        pltpu.VMEM((NI, 128), jnp.int32),
        pltpu.SemaphoreType.DMA,
        pltpu.SemaphoreType.DMA,
    ],
)
def _sc_hist(v_hbm, cnt_out, sum_out,
             cnt_v, sum_v, pb0, pb1, jtab, sp0, sp1):
    wid = lax.axis_index("s") * NC + lax.axis_index("c")
    row_base = wid * RPW

    zeros = jnp.zeros((L,), jnp.float32)
    ones = jnp.full((L,), 1.0, jnp.float32)

    def zbody(r, carry):
        for cc in range(C_SC // L):
            cnt_v[r, pl.ds(cc * L, L)] = zeros
            sum_v[r, pl.ds(cc * L, L)] = zeros
        return carry

    lax.fori_loop(0, NBINS, zbody, 0)

    iota16 = lax.iota(jnp.int32, L)
    for ct in range(NT):
        cvs = [iota16 + (ct * 128 + gg * L) for gg in range(128 // L)]

        def jbody(r, carry, ct=ct, cvs=cvs):
            for gg in range(128 // L):
                jtab[ct * CR + r, pl.ds(gg * L, L)] = cvs[gg]
            return carry

        lax.fori_loop(0, CR, jbody, 0)

    def copies(pb, sp, c):
        r0 = row_base + c * CR
        return [
            pltpu.make_async_copy(
                v_hbm.at[pl.ds(r0, CR), pl.ds(ct * 128, 128)],
                pb.at[pl.ds(ct * CR, CR), :], sp)
            for ct in range(NT)
        ]

    def start(pb, sp, c):
        for cp in copies(pb, sp, c):
            cp.start()

    def wait(pb, sp, c):
        for cp in copies(pb, sp, c):
            cp.wait()

    def process(pb):
        def ibody(i, carry):
            for half in range(128 // (K * L)):
                ks = range(K)
                base = half * K * L
                v = [pb[i, pl.ds(base + k * L, L)] for k in ks]
                j = [jtab[i, pl.ds(base + k * L, L)] for k in ks]
                vi = [lax.bitcast_convert_type(v[k], jnp.int32) for k in ks]
                av = [jnp.abs(v[k]) for k in ks]
                nav = [lax.bitcast_convert_type(vi[k] | jnp.int32(-2**31),
                                                jnp.float32) for k in ks]
                e = [jnp.exp(nav[k]) for k in ks]
                d = [1.0 + e[k] for k in ks]
                r = [1.0 / d[k] for k in ks]
                rr = [r[k] * float(NBINS) for k in ks]
                g30 = [jnp.where(v[k] >= 0.0, float(NBINS) - rr[k], rr[k])
                       for k in ks]
                g30 = [jnp.minimum(g30[k], float(NBINS) - 0.5) for k in ks]
                b = [g30[k].astype(jnp.int32) for k in ks]
                relu = [(av[k] - v[k]) * 0.5 for k in ks]
                p = [jnp.full((L,), _LOG1P[4], jnp.float32)] * K
                for cc in _LOG1P[3::-1]:
                    p = [p[k] * e[k] + cc for k in ks]
                bce = [relu[k] + p[k] for k in ks]
                for k in ks:
                    plsc.addupdate_scatter(cnt_v, [b[k], j[k]], ones)
                    plsc.addupdate_scatter(sum_v, [b[k], j[k]], bce[k])
            return carry

        lax.fori_loop(0, NI, ibody, 0, unroll=2)

    start(pb0, sp0, 0)
    start(pb1, sp1, 1)

    def pbody(pi, carry):
        c0 = 2 * pi
        wait(pb0, sp0, c0)
        process(pb0)

        @pl.when(c0 + 2 < NCH)
        def _():
            start(pb0, sp0, c0 + 2)

        wait(pb1, sp1, c0 + 1)
        process(pb1)

        @pl.when(c0 + 3 < NCH)
        def _():
            start(pb1, sp1, c0 + 3)

        return carry

    lax.fori_loop(0, NCH // 2, pbody, 0)

    pltpu.sync_copy(cnt_v, cnt_out.at[wid])
    pltpu.sync_copy(sum_v, sum_out.at[wid])


def _tc_hist_body(v_ref, cnt_ref, sum_ref):
    v = v_ref[...]
    bce = jnp.maximum(-v, 0.0) + jnp.log1p(jnp.exp(-jnp.abs(v)))
    prev_c = jnp.full((v.shape[1],), float(v.shape[0]), jnp.float32)
    prev_s = jnp.sum(bce, axis=0)
    cs, ss = [], []
    for th in _THRESH:
        m = (v <= th).astype(jnp.float32)
        ci = jnp.sum(m, axis=0)
        si = jnp.sum(m * bce, axis=0)
        cs.append(prev_c - ci)
        ss.append(prev_s - si)
        prev_c, prev_s = ci, si
    cs.append(prev_c)
    ss.append(prev_s)
    cnt_ref[0] = jnp.stack(cs)
    sum_ref[0] = jnp.stack(ss)


def _make_tc_hist(nrows, width, grid, row_off):
    return pl.pallas_call(
        _tc_hist_body,
        grid=(grid,),
        in_specs=[pl.BlockSpec((nrows, width), lambda i: (row_off + i, 0))],
        out_specs=[
            pl.BlockSpec((1, NBINS, width), lambda i: (i, 0, 0)),
            pl.BlockSpec((1, NBINS, width), lambda i: (i, 0, 0)),
        ],
        out_shape=[
            jax.ShapeDtypeStruct((grid, NBINS, width), jnp.float32),
            jax.ShapeDtypeStruct((grid, NBINS, width), jnp.float32),
        ],
    )


_tc_main = _make_tc_hist(BR, COLS, G_TC, R_SC // BR)
_tc_strip = _make_tc_hist(BS, C_ST, G_ST, 0)


def _finalize_body(csc_ref, ssc_ref, cm_ref, sm_ref, cst_ref, sst_ref, out_ref):
    csc = jnp.sum(csc_ref[...], axis=0)
    ssc = jnp.sum(ssc_ref[...], axis=0)
    c = jnp.sum(cm_ref[...], axis=0)
    s = jnp.sum(sm_ref[...], axis=0)
    cst = jnp.sum(cst_ref[...], axis=0)
    sst = jnp.sum(sst_ref[...], axis=0)
    c = c + jnp.concatenate([csc, cst], axis=1)
    s = s + jnp.concatenate([ssc, sst], axis=1)
    nz = c > 0.0
    w = jnp.where(nz, TOT / (0.25 * jnp.maximum(c, 1.0)), 0.0)
    n = jnp.maximum(jnp.sum(nz.astype(jnp.float32), axis=0), 1.0)
    colsum = jnp.sum(s * w, axis=0)
    loss = jnp.sum(colsum / n) / float(ROWS * COLS)
    out_ref[...] = jnp.broadcast_to(loss, (1, 1))


_finalize = pl.pallas_call(
    _finalize_body,
    out_shape=jax.ShapeDtypeStruct((1, 1), jnp.float32),
)


def kernel(preds, targets):
    v = preds * (1.0 - 2.0 * targets.astype(jnp.float32))
    cnt_sc, sum_sc = _sc_hist(v)
    cnt_tc, sum_tc = _tc_main(v)
    cnt_st, sum_st = _tc_strip(v[:R_SC, C_SC:])
    loss = _finalize(cnt_sc, sum_sc, cnt_tc, sum_tc, cnt_st, sum_st)
    return loss[0, 0]
